# baseline (speedup 1.0000x reference)
import functools

import jax
import jax.numpy as jnp
from jax.experimental import pallas as pl

NUM_NODES = 2048
IN_DIM = 128
HIDDEN_DIM = 128
OUT_DIM = 128
HEADS = 4
BATCH = 2
F = HIDDEN_DIM // HEADS
FE = F + 1
JB = 256


def _wext(W):
    K = W.shape[0]
    Wr = W.reshape(K, HEADS, F).transpose(1, 0, 2)
    return jnp.pad(Wr, ((0, 0), (0, 0), (0, 1))).astype(jnp.float32)


def _aext(a):
    return jnp.pad(a, ((0, 0), (0, 1)))[:, :, None].astype(jnp.float32)


def _dot_t(a, b):
    return jax.lax.dot_general(a, b, (((1,), (1,)), ((), ())),
                               preferred_element_type=jnp.float32)


def _normalize(emb):
    return emb / (jnp.sqrt(jnp.sum(emb * emb, axis=1, keepdims=True)) + 1e-12)


def _mask_body(embf_ref, embb_ref, mask_ref):
    adj = _dot_t(_normalize(embb_ref[...]), _normalize(embf_ref[...]))
    mask_ref[...] = (adj > 0.5).astype(jnp.bfloat16)


def _proj_body(x_ref, pW_ref, pb_ref, h_ref):
    x = x_ref[...].reshape(BATCH * NUM_NODES, IN_DIM)
    h = jnp.dot(x, pW_ref[...], preferred_element_type=jnp.float32) \
        + pb_ref[...][None, :]
    h_ref[...] = h.reshape(BATCH, NUM_NODES, IN_DIM)


def _perhead_body(h_ref, We_ref, As_ref, Ad_ref,
                  u1_ref, u2_ref, s_ref, negd_ref, d1_ref, d2_ref):
    ones_col = (jnp.arange(FE) == F).astype(jnp.float32)[None, :]
    hb = h_ref[0]
    We = We_ref[0]
    whx = jnp.dot(hb, We, preferred_element_type=jnp.float32) + ones_col
    s_col = jnp.dot(whx, As_ref[0], preferred_element_type=jnp.float32)
    d_col = jnp.dot(whx, Ad_ref[0], preferred_element_type=jnp.float32)
    smax = jnp.max(s_col)
    dmax = jnp.max(d_col)
    u1_ref[0, 0] = (jnp.exp(s_col - smax) * whx).astype(jnp.bfloat16)
    u2_ref[0, 0] = (jnp.exp(0.2 * s_col - smax) * whx).astype(jnp.bfloat16)
    s_ref[0, 0, 0] = s_col.reshape(1, NUM_NODES)[0]
    negd_ref[0, 0, 0] = (-d_col).reshape(1, NUM_NODES)[0]
    d1_ref[0, 0, 0] = jnp.exp(d_col - dmax).reshape(1, NUM_NODES)[0]
    d2_ref[0, 0, 0] = jnp.exp(0.2 * d_col - dmax).reshape(1, NUM_NODES)[0]


def _perhead(h, We, As, Ad):
    N = NUM_NODES
    K = We.shape[1]
    return pl.pallas_call(
        _perhead_body,
        grid=(BATCH * HEADS,),
        in_specs=[
            pl.BlockSpec((1, N, K), lambda g: (g // HEADS, 0, 0)),
            pl.BlockSpec((1, K, FE), lambda g: (g % HEADS, 0, 0)),
            pl.BlockSpec((1, FE, 1), lambda g: (g % HEADS, 0, 0)),
            pl.BlockSpec((1, FE, 1), lambda g: (g % HEADS, 0, 0)),
        ],
        out_specs=(
            pl.BlockSpec((1, 1, N, FE), lambda g: (g // HEADS, g % HEADS, 0, 0)),
            pl.BlockSpec((1, 1, N, FE), lambda g: (g // HEADS, g % HEADS, 0, 0)),
            pl.BlockSpec((1, 1, 1, N), lambda g: (g // HEADS, g % HEADS, 0, 0)),
            pl.BlockSpec((1, 1, 1, N), lambda g: (g // HEADS, g % HEADS, 0, 0)),
            pl.BlockSpec((1, 1, 1, N), lambda g: (g // HEADS, g % HEADS, 0, 0)),
            pl.BlockSpec((1, 1, 1, N), lambda g: (g // HEADS, g % HEADS, 0, 0)),
        ),
        out_shape=(
            jax.ShapeDtypeStruct((BATCH, HEADS, N, FE), jnp.bfloat16),
            jax.ShapeDtypeStruct((BATCH, HEADS, N, FE), jnp.bfloat16),
            jax.ShapeDtypeStruct((BATCH, HEADS, 1, N), jnp.float32),
            jax.ShapeDtypeStruct((BATCH, HEADS, 1, N), jnp.float32),
            jax.ShapeDtypeStruct((BATCH, HEADS, 1, N), jnp.float32),
            jax.ShapeDtypeStruct((BATCH, HEADS, 1, N), jnp.float32),
        ),
    )(h, We, As, Ad)


def _attn_body(mask_ref, u1_ref, u2_ref, s_ref, negd_ref, d1_ref, d2_ref,
               bias_ref, out_ref, *, relu):
    maskb = mask_ref[...]
    zero = jnp.zeros((), jnp.bfloat16)
    for b in range(BATCH):
        outs = []
        for h in range(HEADS):
            s = s_ref[b, h, 0, :]
            negd = negd_ref[b, h, 0, :]
            cond = s[None, :] >= negd[:, None]
            P = jnp.where(cond, maskb, zero)
            Q = maskb - P
            R1 = jnp.dot(P, u1_ref[b, h],
                         preferred_element_type=jnp.float32)
            R2 = jnp.dot(Q, u2_ref[b, h],
                         preferred_element_type=jnp.float32)
            R = d1_ref[b, h, 0, :][:, None] * R1 \
                + d2_ref[b, h, 0, :][:, None] * R2
            outs.append(R[:, :F] / R[:, F:FE])
        o = jnp.concatenate(outs, axis=1) + bias_ref[...][None, :]
        if relu:
            o = jnp.maximum(o, 0.0)
        out_ref[b] = o


def _attn_layer(mask, u1, u2, s, negd, d1, d2, bias, relu):
    N, HD = NUM_NODES, HIDDEN_DIM
    full_bhn = pl.BlockSpec((BATCH, HEADS, 1, N), lambda j: (0, 0, 0, 0))
    blk_bhn = pl.BlockSpec((BATCH, HEADS, 1, JB), lambda j: (0, 0, 0, j))
    return pl.pallas_call(
        functools.partial(_attn_body, relu=relu),
        grid=(N // JB,),
        in_specs=[
            pl.BlockSpec((JB, N), lambda j: (j, 0)),
            pl.BlockSpec((BATCH, HEADS, N, FE), lambda j: (0, 0, 0, 0)),
            pl.BlockSpec((BATCH, HEADS, N, FE), lambda j: (0, 0, 0, 0)),
            full_bhn, blk_bhn, blk_bhn, blk_bhn,
            pl.BlockSpec((HD,), lambda j: (0,)),
        ],
        out_specs=pl.BlockSpec((BATCH, JB, HD), lambda j: (0, j, 0)),
        out_shape=jax.ShapeDtypeStruct((BATCH, N, HD), jnp.float32),
    )(mask, u1, u2, s, negd, d1, d2, bias)


def kernel(x, embedding, proj_W, proj_b, W1, a1_src, a1_dst, b1,
           W2, a2_src, a2_dst, b2):
    N = NUM_NODES
    W1e, W2e = _wext(W1), _wext(W2)
    A1s, A1d = _aext(a1_src), _aext(a1_dst)
    A2s, A2d = _aext(a2_src), _aext(a2_dst)

    mask = pl.pallas_call(
        _mask_body,
        grid=(N // JB,),
        in_specs=[
            pl.BlockSpec((N, IN_DIM), lambda j: (0, 0)),
            pl.BlockSpec((JB, IN_DIM), lambda j: (j, 0)),
        ],
        out_specs=pl.BlockSpec((JB, N), lambda j: (j, 0)),
        out_shape=jax.ShapeDtypeStruct((N, N), jnp.bfloat16),
    )(embedding, embedding)

    h = pl.pallas_call(
        _proj_body,
        out_shape=jax.ShapeDtypeStruct((BATCH, N, IN_DIM), jnp.float32),
    )(x, proj_W, proj_b)

    u1, u2, s, negd, d1, d2 = _perhead(h, W1e, A1s, A1d)
    h1 = _attn_layer(mask, u1, u2, s, negd, d1, d2, b1, relu=True)

    u1, u2, s, negd, d1, d2 = _perhead(h1, W2e, A2s, A2d)
    return _attn_layer(mask, u1, u2, s, negd, d1, d2, b2, relu=False)


# speedup vs baseline: 1.3058x; 1.3058x over previous
import functools

import jax
import jax.numpy as jnp
from jax.experimental import pallas as pl

NUM_NODES = 2048
IN_DIM = 128
HIDDEN_DIM = 128
OUT_DIM = 128
HEADS = 4
BATCH = 2
F = HIDDEN_DIM // HEADS
FE = F + 1
JB = 256


def _wext(W):
    K = W.shape[0]
    Wr = W.reshape(K, HEADS, F).transpose(1, 0, 2)
    return jnp.pad(Wr, ((0, 0), (0, 0), (0, 1))).astype(jnp.float32)


def _dot_t(a, b):
    return jax.lax.dot_general(a, b, (((1,), (1,)), ((), ())),
                               preferred_element_type=jnp.float32)


def _dot_ff(a, b):
    return jax.lax.dot_general(a, b, (((0,), (0,)), ((), ())),
                               preferred_element_type=jnp.float32)


def _normalize(emb):
    return emb / (jnp.sqrt(jnp.sum(emb * emb, axis=1, keepdims=True)) + 1e-12)


def _mask_body(embf_ref, embb_ref, mask_ref):
    adj = _dot_t(_normalize(embf_ref[...]), _normalize(embb_ref[...]))
    mask_ref[...] = (adj > 0.5).astype(jnp.bfloat16)


def _proj_body(x_ref, pW_ref, pb_ref, h_ref):
    x = x_ref[...].reshape(BATCH * NUM_NODES, IN_DIM)
    h = jnp.dot(x, pW_ref[...], preferred_element_type=jnp.float32) \
        + pb_ref[...][None, :]
    h_ref[...] = h.reshape(BATCH, NUM_NODES, IN_DIM)


def _perhead_body(h_ref, We_ref, As_ref, Adr_ref,
                  u1_ref, u2_ref, sc_ref, negd_ref, d1_ref, d2_ref,
                  *, h_cols):
    ones_col = (jnp.arange(FE) == F).astype(jnp.float32)[None, :]
    if h_cols:
        whx = _dot_ff(h_ref[0], We_ref[0])
    else:
        whx = jnp.dot(h_ref[0], We_ref[0],
                      preferred_element_type=jnp.float32)
    whx = whx + ones_col
    s_col = jnp.dot(whx, As_ref[0], preferred_element_type=jnp.float32)
    d_row = _dot_t(Adr_ref[0], whx)
    smax = jnp.max(s_col)
    dmax = jnp.max(d_row)
    u1_ref[0, 0] = (jnp.exp(s_col - smax) * whx).astype(jnp.bfloat16)
    u2_ref[0, 0] = (jnp.exp(0.2 * s_col - smax) * whx).astype(jnp.bfloat16)
    sc_ref[0, 0] = s_col.astype(jnp.bfloat16)
    negd_ref[0, 0] = (-d_row).astype(jnp.bfloat16)
    d1_ref[0, 0] = jnp.exp(d_row - dmax)
    d2_ref[0, 0] = jnp.exp(0.2 * d_row - dmax)


def _perhead(h, We, As, Adr, h_cols):
    N = NUM_NODES
    K = We.shape[1]
    bh = lambda g: (g // HEADS, g % HEADS, 0, 0)
    hspec = (pl.BlockSpec((1, K, N), lambda g: (g // HEADS, 0, 0)) if h_cols
             else pl.BlockSpec((1, N, K), lambda g: (g // HEADS, 0, 0)))
    return pl.pallas_call(
        functools.partial(_perhead_body, h_cols=h_cols),
        grid=(BATCH * HEADS,),
        in_specs=[
            hspec,
            pl.BlockSpec((1, K, FE), lambda g: (g % HEADS, 0, 0)),
            pl.BlockSpec((1, FE, 1), lambda g: (g % HEADS, 0, 0)),
            pl.BlockSpec((1, 1, FE), lambda g: (g % HEADS, 0, 0)),
        ],
        out_specs=(
            pl.BlockSpec((1, 1, N, FE), bh),
            pl.BlockSpec((1, 1, N, FE), bh),
            pl.BlockSpec((1, 1, N, 1), bh),
            pl.BlockSpec((1, 1, 1, N), bh),
            pl.BlockSpec((1, 1, 1, N), bh),
            pl.BlockSpec((1, 1, 1, N), bh),
        ),
        out_shape=(
            jax.ShapeDtypeStruct((BATCH, HEADS, N, FE), jnp.bfloat16),
            jax.ShapeDtypeStruct((BATCH, HEADS, N, FE), jnp.bfloat16),
            jax.ShapeDtypeStruct((BATCH, HEADS, N, 1), jnp.bfloat16),
            jax.ShapeDtypeStruct((BATCH, HEADS, 1, N), jnp.bfloat16),
            jax.ShapeDtypeStruct((BATCH, HEADS, 1, N), jnp.float32),
            jax.ShapeDtypeStruct((BATCH, HEADS, 1, N), jnp.float32),
        ),
    )(h, We, As, Adr)


def _attn_body(mask_ref, u1_ref, u2_ref, sc_ref, negd_ref, d1_ref, d2_ref,
               bias_ref, out_ref, *, relu):
    maskT = mask_ref[...]
    zero = jnp.zeros((), jnp.bfloat16)
    for b in range(BATCH):
        outs = []
        for h in range(HEADS):
            sc = sc_ref[b, h]
            negd = negd_ref[b, h]
            cond = sc >= negd
            P = jnp.where(cond, maskT, zero)
            Q = maskT - P
            A = _dot_ff(u1_ref[b, h], P)
            Bm = _dot_ff(u2_ref[b, h], Q)
            R = d1_ref[b, h] * A + d2_ref[b, h] * Bm
            outs.append(R[:F, :] / R[F:FE, :])
        o = jnp.concatenate(outs, axis=0) + bias_ref[...]
        if relu:
            o = jnp.maximum(o, 0.0)
        out_ref[b] = o


def _attn_layer(mask, u1, u2, sc, negd, d1, d2, bias_col, relu):
    N, HD = NUM_NODES, HIDDEN_DIM
    full_ufe = pl.BlockSpec((BATCH, HEADS, N, FE), lambda j: (0, 0, 0, 0))
    blk_row = pl.BlockSpec((BATCH, HEADS, 1, JB), lambda j: (0, 0, 0, j))
    return pl.pallas_call(
        functools.partial(_attn_body, relu=relu),
        grid=(N // JB,),
        in_specs=[
            pl.BlockSpec((N, JB), lambda j: (0, j)),
            full_ufe, full_ufe,
            pl.BlockSpec((BATCH, HEADS, N, 1), lambda j: (0, 0, 0, 0)),
            blk_row, blk_row, blk_row,
            pl.BlockSpec((HD, 1), lambda j: (0, 0)),
        ],
        out_specs=pl.BlockSpec((BATCH, HD, JB), lambda j: (0, 0, j)),
        out_shape=jax.ShapeDtypeStruct((BATCH, HD, N), jnp.float32),
    )(mask, u1, u2, sc, negd, d1, d2, bias_col)


def kernel(x, embedding, proj_W, proj_b, W1, a1_src, a1_dst, b1,
           W2, a2_src, a2_dst, b2):
    N = NUM_NODES
    W1e, W2e = _wext(W1), _wext(W2)
    A1s = jnp.pad(a1_src, ((0, 0), (0, 1)))[:, :, None]
    A2s = jnp.pad(a2_src, ((0, 0), (0, 1)))[:, :, None]
    A1d = jnp.pad(a1_dst, ((0, 0), (0, 1)))[:, None, :]
    A2d = jnp.pad(a2_dst, ((0, 0), (0, 1)))[:, None, :]

    mask = pl.pallas_call(
        _mask_body,
        grid=(N // JB,),
        in_specs=[
            pl.BlockSpec((N, IN_DIM), lambda j: (0, 0)),
            pl.BlockSpec((JB, IN_DIM), lambda j: (j, 0)),
        ],
        out_specs=pl.BlockSpec((N, JB), lambda j: (0, j)),
        out_shape=jax.ShapeDtypeStruct((N, N), jnp.bfloat16),
    )(embedding, embedding)

    h = pl.pallas_call(
        _proj_body,
        out_shape=jax.ShapeDtypeStruct((BATCH, N, IN_DIM), jnp.float32),
    )(x, proj_W, proj_b)

    u1, u2, sc, negd, d1, d2 = _perhead(h, W1e, A1s, A1d, h_cols=False)
    h1 = _attn_layer(mask, u1, u2, sc, negd, d1, d2, b1[:, None], relu=True)

    u1, u2, sc, negd, d1, d2 = _perhead(h1, W2e, A2s, A2d, h_cols=True)
    out_t = _attn_layer(mask, u1, u2, sc, negd, d1, d2, b2[:, None], relu=False)
    return jnp.swapaxes(out_t, 1, 2)


# speedup vs baseline: 1.8225x; 1.3958x over previous
import functools

import jax
import jax.numpy as jnp
from jax.experimental import pallas as pl

NUM_NODES = 2048
IN_DIM = 128
HIDDEN_DIM = 128
OUT_DIM = 128
HEADS = 4
BATCH = 2
F = HIDDEN_DIM // HEADS
FE = F + 1
JB = 256


def _wext_t(W):
    K = W.shape[0]
    Wr = W.reshape(K, HEADS, F).transpose(1, 2, 0)
    return jnp.pad(Wr, ((0, 0), (0, 1), (0, 0))).astype(jnp.float32)


def _dot_t(a, b):
    return jax.lax.dot_general(a, b, (((1,), (1,)), ((), ())),
                               preferred_element_type=jnp.float32)


def _normalize(emb):
    return emb / (jnp.sqrt(jnp.sum(emb * emb, axis=1, keepdims=True)) + 1e-12)


def _mask_body(embf_ref, embb_ref, mask_ref):
    adj = _dot_t(_normalize(embf_ref[...]), _normalize(embb_ref[...]))
    mask_ref[...] = (adj > 0.5).astype(jnp.bfloat16)


def _proj_body(x_ref, pW_ref, pb_ref, h_ref):
    x = x_ref[...].reshape(BATCH * NUM_NODES, IN_DIM)
    h = jnp.dot(x, pW_ref[...], preferred_element_type=jnp.float32) \
        + pb_ref[...][None, :]
    h_ref[...] = h.reshape(BATCH, NUM_NODES, IN_DIM)


def _perhead_body(h_ref, WeT_ref, Asr_ref, Adr_ref, Was_ref,
                  u1_ref, u2_ref, sc_ref, negd_ref, d1_ref, d2_ref):
    ones_row = (jnp.arange(FE) == F).astype(jnp.float32)[:, None]
    hb = h_ref[0]
    whxT = _dot_t(WeT_ref[0], hb) + ones_row
    s_row = jnp.dot(Asr_ref[0], whxT,
                    preferred_element_type=jnp.float32)
    d_row = jnp.dot(Adr_ref[0], whxT,
                    preferred_element_type=jnp.float32)
    s_col = jnp.dot(hb, Was_ref[0],
                    preferred_element_type=jnp.float32)
    smax = jnp.max(s_row)
    dmax = jnp.max(d_row)
    u1_ref[0, 0] = (jnp.exp(s_row - smax) * whxT).astype(jnp.bfloat16)
    u2_ref[0, 0] = (jnp.exp(0.2 * s_row - smax) * whxT).astype(jnp.bfloat16)
    sc_ref[0, 0] = s_col
    negd_ref[0, 0] = -d_row
    d1_ref[0, 0] = jnp.exp(d_row - dmax)
    d2_ref[0, 0] = jnp.exp(0.2 * d_row - dmax)


def _perhead(h, WeT, Asr, Adr, Was):
    N = NUM_NODES
    K = WeT.shape[2]
    bh = lambda g: (g // HEADS, g % HEADS, 0, 0)
    hw = lambda g: (g % HEADS, 0, 0)
    return pl.pallas_call(
        _perhead_body,
        grid=(BATCH * HEADS,),
        in_specs=[
            pl.BlockSpec((1, N, K), lambda g: (g // HEADS, 0, 0)),
            pl.BlockSpec((1, FE, K), hw),
            pl.BlockSpec((1, 1, FE), hw),
            pl.BlockSpec((1, 1, FE), hw),
            pl.BlockSpec((1, K, 1), hw),
        ],
        out_specs=(
            pl.BlockSpec((1, 1, FE, N), bh),
            pl.BlockSpec((1, 1, FE, N), bh),
            pl.BlockSpec((1, 1, N, 1), bh),
            pl.BlockSpec((1, 1, 1, N), bh),
            pl.BlockSpec((1, 1, 1, N), bh),
            pl.BlockSpec((1, 1, 1, N), bh),
        ),
        out_shape=(
            jax.ShapeDtypeStruct((BATCH, HEADS, FE, N), jnp.bfloat16),
            jax.ShapeDtypeStruct((BATCH, HEADS, FE, N), jnp.bfloat16),
            jax.ShapeDtypeStruct((BATCH, HEADS, N, 1), jnp.float32),
            jax.ShapeDtypeStruct((BATCH, HEADS, 1, N), jnp.float32),
            jax.ShapeDtypeStruct((BATCH, HEADS, 1, N), jnp.float32),
            jax.ShapeDtypeStruct((BATCH, HEADS, 1, N), jnp.float32),
        ),
    )(h, WeT, Asr, Adr, Was)


def _attn_body(mask_ref, u1_ref, u2_ref, sc_ref, negd_ref, d1_ref, d2_ref,
               bias_ref, out_ref, *, relu):
    maskT = mask_ref[...]
    zero = jnp.zeros((), jnp.bfloat16)
    for b in range(BATCH):
        outs = []
        for h in range(HEADS):
            sc = sc_ref[b, h]
            negd = negd_ref[b, h]
            cond = sc >= negd
            P = jnp.where(cond, maskT, zero)
            Q = maskT - P
            A = jnp.dot(u1_ref[b, h], P,
                        preferred_element_type=jnp.float32)
            Bm = jnp.dot(u2_ref[b, h], Q,
                         preferred_element_type=jnp.float32)
            R = d1_ref[b, h] * A + d2_ref[b, h] * Bm
            outs.append(R[:F, :] / R[F:FE, :])
        o = jnp.concatenate(outs, axis=0) + bias_ref[...]
        if relu:
            o = jnp.maximum(o, 0.0)
        out_ref[b] = o


def _attn_layer(mask, u1, u2, sc, negd, d1, d2, bias_col, relu):
    N, HD = NUM_NODES, HIDDEN_DIM
    full_ut = pl.BlockSpec((BATCH, HEADS, FE, N), lambda j: (0, 0, 0, 0))
    blk_row = pl.BlockSpec((BATCH, HEADS, 1, JB), lambda j: (0, 0, 0, j))
    return pl.pallas_call(
        functools.partial(_attn_body, relu=relu),
        grid=(N // JB,),
        in_specs=[
            pl.BlockSpec((N, JB), lambda j: (0, j)),
            full_ut, full_ut,
            pl.BlockSpec((BATCH, HEADS, N, 1), lambda j: (0, 0, 0, 0)),
            blk_row, blk_row, blk_row,
            pl.BlockSpec((HD, 1), lambda j: (0, 0)),
        ],
        out_specs=pl.BlockSpec((BATCH, HD, JB), lambda j: (0, 0, j)),
        out_shape=jax.ShapeDtypeStruct((BATCH, HD, N), jnp.float32),
    )(mask, u1, u2, sc, negd, d1, d2, bias_col)


def kernel(x, embedding, proj_W, proj_b, W1, a1_src, a1_dst, b1,
           W2, a2_src, a2_dst, b2):
    N = NUM_NODES
    W1eT, W2eT = _wext_t(W1), _wext_t(W2)
    A1s = jnp.pad(a1_src, ((0, 0), (0, 1)))[:, None, :]
    A2s = jnp.pad(a2_src, ((0, 0), (0, 1)))[:, None, :]
    A1d = jnp.pad(a1_dst, ((0, 0), (0, 1)))[:, None, :]
    A2d = jnp.pad(a2_dst, ((0, 0), (0, 1)))[:, None, :]
    W1as = jnp.einsum('khf,hf->hk', W1.reshape(IN_DIM, HEADS, F),
                      a1_src)[:, :, None]
    W2as = jnp.einsum('khf,hf->hk', W2.reshape(HIDDEN_DIM, HEADS, F),
                      a2_src)[:, :, None]

    mask = pl.pallas_call(
        _mask_body,
        grid=(N // JB,),
        in_specs=[
            pl.BlockSpec((N, IN_DIM), lambda j: (0, 0)),
            pl.BlockSpec((JB, IN_DIM), lambda j: (j, 0)),
        ],
        out_specs=pl.BlockSpec((N, JB), lambda j: (0, j)),
        out_shape=jax.ShapeDtypeStruct((N, N), jnp.bfloat16),
    )(embedding, embedding)

    h = pl.pallas_call(
        _proj_body,
        out_shape=jax.ShapeDtypeStruct((BATCH, N, IN_DIM), jnp.float32),
    )(x, proj_W, proj_b)

    u1, u2, sc, negd, d1, d2 = _perhead(h, W1eT, A1s, A1d, W1as)
    h1t = _attn_layer(mask, u1, u2, sc, negd, d1, d2, b1[:, None], relu=True)

    h1 = jnp.swapaxes(h1t, 1, 2)
    u1, u2, sc, negd, d1, d2 = _perhead(h1, W2eT, A2s, A2d, W2as)
    out_t = _attn_layer(mask, u1, u2, sc, negd, d1, d2, b2[:, None], relu=False)
    return jnp.swapaxes(out_t, 1, 2)


# speedup vs baseline: 1.8308x; 1.0045x over previous
import functools

import jax
import jax.numpy as jnp
from jax.experimental import pallas as pl

NUM_NODES = 2048
IN_DIM = 128
HIDDEN_DIM = 128
OUT_DIM = 128
HEADS = 4
BATCH = 2
F = HIDDEN_DIM // HEADS
FE = F + 1
JB = 256


def _wext_t(W):
    K = W.shape[0]
    Wr = W.reshape(K, HEADS, F).transpose(1, 2, 0)
    return jnp.pad(Wr, ((0, 0), (0, 1), (0, 0))).astype(jnp.float32)


def _dot_t(a, b):
    return jax.lax.dot_general(a, b, (((1,), (1,)), ((), ())),
                               preferred_element_type=jnp.float32)


def _normalize(emb):
    return emb / (jnp.sqrt(jnp.sum(emb * emb, axis=1, keepdims=True)) + 1e-12)


def _mask_body(embf_ref, embb_ref, mask_ref):
    adj = _dot_t(_normalize(embf_ref[...]), _normalize(embb_ref[...]))
    mask_ref[...] = (adj > 0.5).astype(jnp.bfloat16)


def _perhead_body(h_ref, WeT_ref, Asr_ref, Adr_ref, Was1_ref,
                  *rest, project):
    if project:
        (pW_ref, pb_ref, u1_ref, u2_ref, sc_ref, nd_ref, d1_ref,
         d2_ref) = rest
        hb = jnp.dot(h_ref[0], pW_ref[...],
                     preferred_element_type=jnp.float32) + pb_ref[...][None, :]
    else:
        u1_ref, u2_ref, sc_ref, nd_ref, d1_ref, d2_ref = rest
        hb = h_ref[0]
    ones_row = (jnp.arange(FE) == F).astype(jnp.float32)[:, None]
    whxT = _dot_t(WeT_ref[0], hb) + ones_row
    s_row = jnp.dot(Asr_ref[0], whxT,
                    preferred_element_type=jnp.float32)
    d_row = jnp.dot(Adr_ref[0], whxT,
                    preferred_element_type=jnp.float32)
    smax = jnp.max(s_row)
    dmax = jnp.max(d_row)
    u1_ref[0, 0] = (jnp.exp(s_row - smax) * whxT).astype(jnp.bfloat16)
    u2_ref[0, 0] = (jnp.exp(0.2 * s_row - smax) * whxT).astype(jnp.bfloat16)
    sc_ref[0, 0] = jnp.dot(hb, Was1_ref[0],
                           preferred_element_type=jnp.float32)
    nd_ref[0, 0] = -d_row
    d1_ref[0, 0] = jnp.exp(d_row - dmax)
    d2_ref[0, 0] = jnp.exp(0.2 * d_row - dmax)


def _perhead(h, WeT, Asr, Adr, Was1, proj=None):
    N = NUM_NODES
    K = h.shape[2]
    bh = lambda g: (g // HEADS, g % HEADS, 0, 0)
    hw = lambda g: (g % HEADS, 0, 0)
    extra_in, extra_specs = (), ()
    if proj is not None:
        pW, pb = proj
        extra_in = (pW, pb)
        extra_specs = (pl.BlockSpec((K, K), lambda g: (0, 0)),
                       pl.BlockSpec((K,), lambda g: (0,)))
    return pl.pallas_call(
        functools.partial(_perhead_body, project=proj is not None),
        grid=(BATCH * HEADS,),
        in_specs=[
            pl.BlockSpec((1, N, K), lambda g: (g // HEADS, 0, 0)),
            pl.BlockSpec((1, FE, K), hw),
            pl.BlockSpec((1, 1, FE), hw),
            pl.BlockSpec((1, 1, FE), hw),
            pl.BlockSpec((1, K, 1), hw),
            *extra_specs,
        ],
        out_specs=(
            pl.BlockSpec((1, 1, FE, N), bh),
            pl.BlockSpec((1, 1, FE, N), bh),
            pl.BlockSpec((1, 1, N, 1), bh),
            pl.BlockSpec((1, 1, 1, N), bh),
            pl.BlockSpec((1, 1, 1, N), bh),
            pl.BlockSpec((1, 1, 1, N), bh),
        ),
        out_shape=(
            jax.ShapeDtypeStruct((BATCH, HEADS, FE, N), jnp.bfloat16),
            jax.ShapeDtypeStruct((BATCH, HEADS, FE, N), jnp.bfloat16),
            jax.ShapeDtypeStruct((BATCH, HEADS, N, 1), jnp.float32),
            jax.ShapeDtypeStruct((BATCH, HEADS, 1, N), jnp.float32),
            jax.ShapeDtypeStruct((BATCH, HEADS, 1, N), jnp.float32),
            jax.ShapeDtypeStruct((BATCH, HEADS, 1, N), jnp.float32),
        ),
    )(h, WeT, Asr, Adr, Was1, *extra_in)


def _attn_body(mask_ref, u1_ref, u2_ref, sc_ref, nd_ref, d1_ref, d2_ref,
               bias_ref, out_ref, *, relu):
    maskT = mask_ref[...]
    zero = jnp.zeros((), jnp.bfloat16)
    for b in range(BATCH):
        outs = []
        for h in range(HEADS):
            P = jnp.where(sc_ref[b, h] >= nd_ref[b, h], maskT, zero)
            Q = maskT - P
            A = jnp.dot(u1_ref[b, h], P,
                        preferred_element_type=jnp.float32)
            Bm = jnp.dot(u2_ref[b, h], Q,
                         preferred_element_type=jnp.float32)
            R = d1_ref[b, h] * A + d2_ref[b, h] * Bm
            outs.append(R[:F, :] / R[F:FE, :])
        o = jnp.concatenate(outs, axis=0) + bias_ref[...]
        if relu:
            o = jnp.maximum(o, 0.0)
        out_ref[b] = o


def _attn_layer(mask, u1, u2, sc, nd, d1, d2, bias_col, relu):
    N, HD = NUM_NODES, HIDDEN_DIM
    full_ut = pl.BlockSpec((BATCH, HEADS, FE, N), lambda j: (0, 0, 0, 0))
    blk_row = pl.BlockSpec((BATCH, HEADS, 1, JB), lambda j: (0, 0, 0, j))
    return pl.pallas_call(
        functools.partial(_attn_body, relu=relu),
        grid=(N // JB,),
        in_specs=[
            pl.BlockSpec((N, JB), lambda j: (0, j)),
            full_ut, full_ut,
            pl.BlockSpec((BATCH, HEADS, N, 1), lambda j: (0, 0, 0, 0)),
            blk_row,
            blk_row, blk_row,
            pl.BlockSpec((HD, 1), lambda j: (0, 0)),
        ],
        out_specs=pl.BlockSpec((BATCH, HD, JB), lambda j: (0, 0, j)),
        out_shape=jax.ShapeDtypeStruct((BATCH, HD, N), jnp.float32),
    )(mask, u1, u2, sc, nd, d1, d2, bias_col)


def kernel(x, embedding, proj_W, proj_b, W1, a1_src, a1_dst, b1,
           W2, a2_src, a2_dst, b2):
    N = NUM_NODES
    W1eT, W2eT = _wext_t(W1), _wext_t(W2)
    A1s = jnp.pad(a1_src, ((0, 0), (0, 1)))[:, None, :]
    A2s = jnp.pad(a2_src, ((0, 0), (0, 1)))[:, None, :]
    A1d = jnp.pad(a1_dst, ((0, 0), (0, 1)))[:, None, :]
    A2d = jnp.pad(a2_dst, ((0, 0), (0, 1)))[:, None, :]
    W1as = jnp.einsum('khf,hf->hk', W1.reshape(IN_DIM, HEADS, F), a1_src)
    W2as = jnp.einsum('khf,hf->hk', W2.reshape(HIDDEN_DIM, HEADS, F), a2_src)
    W1as1 = W1as[:, :, None]
    W2as1 = W2as[:, :, None]

    mask = pl.pallas_call(
        _mask_body,
        grid=(N // JB,),
        in_specs=[
            pl.BlockSpec((N, IN_DIM), lambda j: (0, 0)),
            pl.BlockSpec((JB, IN_DIM), lambda j: (j, 0)),
        ],
        out_specs=pl.BlockSpec((N, JB), lambda j: (0, j)),
        out_shape=jax.ShapeDtypeStruct((N, N), jnp.bfloat16),
    )(embedding, embedding)

    u1, u2, sc, nd, d1, d2 = _perhead(x, W1eT, A1s, A1d, W1as1,
                                      proj=(proj_W, proj_b))
    h1t = _attn_layer(mask, u1, u2, sc, nd, d1, d2, b1[:, None], relu=True)

    h1 = jnp.swapaxes(h1t, 1, 2)
    u1, u2, sc, nd, d1, d2 = _perhead(h1, W2eT, A2s, A2d, W2as1)
    out_t = _attn_layer(mask, u1, u2, sc, nd, d1, d2, b2[:, None], relu=False)
    return jnp.swapaxes(out_t, 1, 2)


# speedup vs baseline: 1.8371x; 1.0035x over previous
import functools

import jax
import jax.numpy as jnp
from jax.experimental import pallas as pl

NUM_NODES = 2048
IN_DIM = 128
HIDDEN_DIM = 128
OUT_DIM = 128
HEADS = 4
BATCH = 2
F = HIDDEN_DIM // HEADS
FE = F + 1
JB = 256


def _wext_t(W):
    K = W.shape[0]
    Wr = W.reshape(K, HEADS, F).transpose(1, 2, 0)
    return jnp.pad(Wr, ((0, 0), (0, 1), (0, 0))).astype(jnp.float32)


def _dot_t(a, b):
    return jax.lax.dot_general(a, b, (((1,), (1,)), ((), ())),
                               preferred_element_type=jnp.float32)


def _normalize(emb):
    return emb / (jnp.sqrt(jnp.sum(emb * emb, axis=1, keepdims=True)) + 1e-12)


def _mask_body(embf_ref, embb_ref, mask_ref):
    adj = _dot_t(_normalize(embf_ref[...]), _normalize(embb_ref[...]))
    mask_ref[0] = (adj > 0.5).astype(jnp.bfloat16)


def _perhead_body(h_ref, WeT_ref, Asr_ref, Adr_ref, Was1_ref,
                  *rest, project):
    if project:
        (pW_ref, pb_ref, u1_ref, u2_ref, sc_ref, nd_ref, d1_ref,
         d2_ref) = rest
        hb = jnp.dot(h_ref[0], pW_ref[...],
                     preferred_element_type=jnp.float32) + pb_ref[...][None, :]
    else:
        u1_ref, u2_ref, sc_ref, nd_ref, d1_ref, d2_ref = rest
        hb = h_ref[0]
    ones_row = (jnp.arange(FE) == F).astype(jnp.float32)[:, None]
    whxT = _dot_t(WeT_ref[0], hb) + ones_row
    s_row = jnp.dot(Asr_ref[0], whxT,
                    preferred_element_type=jnp.float32)
    d_row = jnp.dot(Adr_ref[0], whxT,
                    preferred_element_type=jnp.float32)
    smax = jnp.max(s_row)
    dmax = jnp.max(d_row)
    u1_ref[0, 0] = (jnp.exp(s_row - smax) * whxT).astype(jnp.bfloat16)
    u2_ref[0, 0] = (jnp.exp(0.2 * s_row - smax) * whxT).astype(jnp.bfloat16)
    sc_ref[0, 0] = jnp.dot(hb, Was1_ref[0],
                           preferred_element_type=jnp.float32)
    nd_ref[0, 0] = -d_row
    d1_ref[0, 0] = jnp.exp(d_row - dmax)
    d2_ref[0, 0] = jnp.exp(0.2 * d_row - dmax)


def _perhead(h, WeT, Asr, Adr, Was1, proj=None):
    N = NUM_NODES
    K = h.shape[2]
    bh = lambda g: (g // HEADS, g % HEADS, 0, 0)
    hw = lambda g: (g % HEADS, 0, 0)
    extra_in, extra_specs = (), ()
    if proj is not None:
        pW, pb = proj
        extra_in = (pW, pb)
        extra_specs = (pl.BlockSpec((K, K), lambda g: (0, 0)),
                       pl.BlockSpec((K,), lambda g: (0,)))
    return pl.pallas_call(
        functools.partial(_perhead_body, project=proj is not None),
        grid=(BATCH * HEADS,),
        in_specs=[
            pl.BlockSpec((1, N, K), lambda g: (g // HEADS, 0, 0)),
            pl.BlockSpec((1, FE, K), hw),
            pl.BlockSpec((1, 1, FE), hw),
            pl.BlockSpec((1, 1, FE), hw),
            pl.BlockSpec((1, K, 1), hw),
            *extra_specs,
        ],
        out_specs=(
            pl.BlockSpec((1, 1, FE, N), bh),
            pl.BlockSpec((1, 1, FE, N), bh),
            pl.BlockSpec((1, 1, N, 1), bh),
            pl.BlockSpec((1, 1, 1, N), bh),
            pl.BlockSpec((1, 1, 1, N), bh),
            pl.BlockSpec((1, 1, 1, N), bh),
        ),
        out_shape=(
            jax.ShapeDtypeStruct((BATCH, HEADS, FE, N), jnp.bfloat16),
            jax.ShapeDtypeStruct((BATCH, HEADS, FE, N), jnp.bfloat16),
            jax.ShapeDtypeStruct((BATCH, HEADS, N, 1), jnp.float32),
            jax.ShapeDtypeStruct((BATCH, HEADS, 1, N), jnp.float32),
            jax.ShapeDtypeStruct((BATCH, HEADS, 1, N), jnp.float32),
            jax.ShapeDtypeStruct((BATCH, HEADS, 1, N), jnp.float32),
        ),
    )(h, WeT, Asr, Adr, Was1, *extra_in)


def _attn_body(mask_ref, u1_ref, u2_ref, sc_ref, nd_ref, d1_ref, d2_ref,
               bias_ref, out_ref, *, relu):
    maskT = mask_ref[0]
    zero = jnp.zeros((), jnp.bfloat16)
    for b in range(BATCH):
        outs = []
        for h in range(HEADS):
            P = jnp.where(sc_ref[b, h] >= nd_ref[b, h], maskT, zero)
            Q = maskT - P
            A = jnp.dot(u1_ref[b, h], P,
                        preferred_element_type=jnp.float32)
            Bm = jnp.dot(u2_ref[b, h], Q,
                         preferred_element_type=jnp.float32)
            R = d1_ref[b, h] * A + d2_ref[b, h] * Bm
            outs.append(R[:F, :] / R[F:FE, :])
        o = jnp.concatenate(outs, axis=0) + bias_ref[...]
        if relu:
            o = jnp.maximum(o, 0.0)
        out_ref[b] = o


def _attn_layer(mask, u1, u2, sc, nd, d1, d2, bias_col, relu):
    N, HD = NUM_NODES, HIDDEN_DIM
    full_ut = pl.BlockSpec((BATCH, HEADS, FE, N), lambda j: (0, 0, 0, 0))
    blk_row = pl.BlockSpec((BATCH, HEADS, 1, JB), lambda j: (0, 0, 0, j))
    return pl.pallas_call(
        functools.partial(_attn_body, relu=relu),
        grid=(N // JB,),
        in_specs=[
            pl.BlockSpec((1, N, JB), lambda j: (j, 0, 0)),
            full_ut, full_ut,
            pl.BlockSpec((BATCH, HEADS, N, 1), lambda j: (0, 0, 0, 0)),
            blk_row,
            blk_row, blk_row,
            pl.BlockSpec((HD, 1), lambda j: (0, 0)),
        ],
        out_specs=pl.BlockSpec((BATCH, HD, JB), lambda j: (0, 0, j)),
        out_shape=jax.ShapeDtypeStruct((BATCH, HD, N), jnp.float32),
    )(mask, u1, u2, sc, nd, d1, d2, bias_col)


def kernel(x, embedding, proj_W, proj_b, W1, a1_src, a1_dst, b1,
           W2, a2_src, a2_dst, b2):
    N = NUM_NODES
    W1eT, W2eT = _wext_t(W1), _wext_t(W2)
    A1s = jnp.pad(a1_src, ((0, 0), (0, 1)))[:, None, :]
    A2s = jnp.pad(a2_src, ((0, 0), (0, 1)))[:, None, :]
    A1d = jnp.pad(a1_dst, ((0, 0), (0, 1)))[:, None, :]
    A2d = jnp.pad(a2_dst, ((0, 0), (0, 1)))[:, None, :]
    W1as = jnp.einsum('khf,hf->hk', W1.reshape(IN_DIM, HEADS, F), a1_src)
    W2as = jnp.einsum('khf,hf->hk', W2.reshape(HIDDEN_DIM, HEADS, F), a2_src)
    W1as1 = W1as[:, :, None]
    W2as1 = W2as[:, :, None]

    mask = pl.pallas_call(
        _mask_body,
        grid=(N // JB,),
        in_specs=[
            pl.BlockSpec((N, IN_DIM), lambda j: (0, 0)),
            pl.BlockSpec((JB, IN_DIM), lambda j: (j, 0)),
        ],
        out_specs=pl.BlockSpec((1, N, JB), lambda j: (j, 0, 0)),
        out_shape=jax.ShapeDtypeStruct((N // JB, N, JB), jnp.bfloat16),
    )(embedding, embedding)

    u1, u2, sc, nd, d1, d2 = _perhead(x, W1eT, A1s, A1d, W1as1,
                                      proj=(proj_W, proj_b))
    h1t = _attn_layer(mask, u1, u2, sc, nd, d1, d2, b1[:, None], relu=True)

    h1 = jnp.swapaxes(h1t, 1, 2)
    u1, u2, sc, nd, d1, d2 = _perhead(h1, W2eT, A2s, A2d, W2as1)
    out_t = _attn_layer(mask, u1, u2, sc, nd, d1, d2, b2[:, None], relu=False)
    return jnp.swapaxes(out_t, 1, 2)


# speedup vs baseline: 2.2932x; 1.2483x over previous
import jax
import jax.numpy as jnp
from jax.experimental import pallas as pl
from jax.experimental.pallas import tpu as pltpu

NUM_NODES = 2048
IN_DIM = 128
HIDDEN_DIM = 128
OUT_DIM = 128
HEADS = 4
BATCH = 2
F = HIDDEN_DIM // HEADS
FE = F + 1
JB = 256
NJ = NUM_NODES // JB


def _wext_t(W):
    K = W.shape[0]
    Wr = W.reshape(K, HEADS, F).transpose(1, 2, 0)
    return jnp.pad(Wr, ((0, 0), (0, 1), (0, 0))).astype(jnp.float32)


def _dot_t(a, b):
    return jax.lax.dot_general(a, b, (((1,), (1,)), ((), ())),
                               preferred_element_type=jnp.float32)


def _normalize(emb):
    return emb / (jnp.sqrt(jnp.sum(emb * emb, axis=1, keepdims=True)) + 1e-12)


def _fused_body(emb_ref, x_ref, pW_ref, pb_ref,
                W1eT_ref, A1s_ref, A1d_ref, W1as_ref,
                W2eT_ref, A2s_ref, A2d_ref,
                b1_ref, b2_ref, out_ref,
                mask_scr, u1_scr, u2_scr, sc_scr, nd_scr, d1_scr, d2_scr,
                h1_scr):
    pid = pl.program_id(0)
    r = pid % NJ
    b = r // HEADS
    h = r % HEADS
    ones_row = (jnp.arange(FE) == F).astype(jnp.float32)[:, None]

    def store_factors(whxT, s_row, d_row, s_col):
        smax = jnp.max(s_row)
        dmax = jnp.max(d_row)
        ib, ih = pl.ds(b, 1), pl.ds(h, 1)
        u1_scr[ib, ih] = (jnp.exp(s_row - smax)
                          * whxT).astype(jnp.bfloat16)[None, None]
        u2_scr[ib, ih] = (jnp.exp(0.2 * s_row - smax)
                          * whxT).astype(jnp.bfloat16)[None, None]
        sc_scr[ib, ih] = s_col[None, None]
        nd_scr[ib, ih] = (-d_row)[None, None]
        d1_scr[ib, ih] = jnp.exp(d_row - dmax)[None, None]
        d2_scr[ib, ih] = jnp.exp(0.2 * d_row - dmax)[None, None]

    def attn(j, first_layer):
        maskT = mask_scr[pl.ds(j, 1)][0]
        zero = jnp.zeros((), jnp.bfloat16)
        lane = pl.ds(j * JB, JB)
        for bb in range(BATCH):
            outs = []
            for hh in range(HEADS):
                cond = sc_scr[bb, hh] >= nd_scr[bb, hh, :, lane]
                P = jnp.where(cond, maskT, zero)
                Q = maskT - P
                A = jnp.dot(u1_scr[bb, hh], P,
                            preferred_element_type=jnp.float32)
                Bm = jnp.dot(u2_scr[bb, hh], Q,
                             preferred_element_type=jnp.float32)
                R = (d1_scr[bb, hh, :, lane] * A
                     + d2_scr[bb, hh, :, lane] * Bm)
                outs.append(R[:F, :] / R[F:FE, :])
            o = jnp.concatenate(outs, axis=0)
            if first_layer:
                o = jnp.maximum(o + b1_ref[...], 0.0)
                h1_scr[pl.ds(j, 1), pl.ds(bb, 1)] = o[None, None]
            else:
                out_ref[bb] = o + b2_ref[...]

    @pl.when(pid < NJ)
    def _phase_mask_prep1():
        nf = _normalize(emb_ref[...])
        nb = _normalize(emb_ref[pl.ds(r * JB, JB), :])
        mask_scr[pl.ds(r, 1)] = (_dot_t(nf, nb) > 0.5).astype(
            jnp.bfloat16)[None]
        hb = jnp.dot(x_ref[pl.ds(b, 1)][0], pW_ref[...],
                     preferred_element_type=jnp.float32) + pb_ref[...][None, :]
        whxT = _dot_t(W1eT_ref[pl.ds(h, 1)][0], hb) + ones_row
        s_row = jnp.dot(A1s_ref[pl.ds(h, 1)][0], whxT,
                        preferred_element_type=jnp.float32)
        d_row = jnp.dot(A1d_ref[pl.ds(h, 1)][0], whxT,
                        preferred_element_type=jnp.float32)
        s_col = jnp.dot(hb, W1as_ref[pl.ds(h, 1)][0],
                        preferred_element_type=jnp.float32)
        store_factors(whxT, s_row, d_row, s_col)

    @pl.when(jnp.logical_and(pid >= NJ, pid < 2 * NJ))
    def _phase_attn1():
        attn(r, True)

    @pl.when(jnp.logical_and(pid >= 2 * NJ, pid < 3 * NJ))
    def _phase_prep2():
        parts = [h1_scr[pl.ds(jj, 1), pl.ds(b, 1)][0, 0] for jj in range(NJ)]
        hbT = jnp.concatenate(parts, axis=1)
        whxT = jnp.dot(W2eT_ref[pl.ds(h, 1)][0], hbT,
                       preferred_element_type=jnp.float32) + ones_row
        s_row = jnp.dot(A2s_ref[pl.ds(h, 1)][0], whxT,
                        preferred_element_type=jnp.float32)
        d_row = jnp.dot(A2d_ref[pl.ds(h, 1)][0], whxT,
                        preferred_element_type=jnp.float32)
        s_col = jnp.reshape(s_row, (NUM_NODES, 1))
        store_factors(whxT, s_row, d_row, s_col)

    @pl.when(pid >= 3 * NJ)
    def _phase_attn2():
        attn(r, False)


def kernel(x, embedding, proj_W, proj_b, W1, a1_src, a1_dst, b1,
           W2, a2_src, a2_dst, b2):
    N = NUM_NODES
    W1eT, W2eT = _wext_t(W1), _wext_t(W2)
    A1s = jnp.pad(a1_src, ((0, 0), (0, 1)))[:, None, :]
    A2s = jnp.pad(a2_src, ((0, 0), (0, 1)))[:, None, :]
    A1d = jnp.pad(a1_dst, ((0, 0), (0, 1)))[:, None, :]
    A2d = jnp.pad(a2_dst, ((0, 0), (0, 1)))[:, None, :]
    W1as = jnp.einsum('khf,hf->hk', W1.reshape(IN_DIM, HEADS, F), a1_src)
    W1as1 = W1as[:, :, None]

    full = lambda *shape: pl.BlockSpec(shape, lambda p: (0,) * len(shape))
    out_t = pl.pallas_call(
        _fused_body,
        grid=(4 * NJ,),
        in_specs=[
            full(N, IN_DIM),
            full(BATCH, N, IN_DIM),
            full(IN_DIM, IN_DIM),
            full(IN_DIM),
            full(HEADS, FE, IN_DIM),
            full(HEADS, 1, FE),
            full(HEADS, 1, FE),
            full(HEADS, IN_DIM, 1),
            full(HEADS, FE, HIDDEN_DIM),
            full(HEADS, 1, FE),
            full(HEADS, 1, FE),
            full(HIDDEN_DIM, 1),
            full(HIDDEN_DIM, 1),
        ],
        out_specs=pl.BlockSpec(
            (BATCH, HIDDEN_DIM, JB),
            lambda p: (0, 0, jnp.maximum(p - 3 * NJ, 0))),
        out_shape=jax.ShapeDtypeStruct((BATCH, HIDDEN_DIM, N), jnp.float32),
        scratch_shapes=[
            pltpu.VMEM((NJ, N, JB), jnp.bfloat16),
            pltpu.VMEM((BATCH, HEADS, FE, N), jnp.bfloat16),
            pltpu.VMEM((BATCH, HEADS, FE, N), jnp.bfloat16),
            pltpu.VMEM((BATCH, HEADS, N, 1), jnp.float32),
            pltpu.VMEM((BATCH, HEADS, 1, N), jnp.float32),
            pltpu.VMEM((BATCH, HEADS, 1, N), jnp.float32),
            pltpu.VMEM((BATCH, HEADS, 1, N), jnp.float32),
            pltpu.VMEM((NJ, BATCH, HIDDEN_DIM, JB), jnp.float32),
        ],
    )(embedding, x, proj_W, proj_b,
      W1eT, A1s, A1d, W1as1,
      W2eT, A2s, A2d,
      b1[:, None], b2[:, None])
    return jnp.swapaxes(out_t, 1, 2)


# speedup vs baseline: 2.3028x; 1.0042x over previous
import jax
import jax.numpy as jnp
from jax.experimental import pallas as pl
from jax.experimental.pallas import tpu as pltpu

NUM_NODES = 2048
IN_DIM = 128
HIDDEN_DIM = 128
OUT_DIM = 128
HEADS = 4
BATCH = 2
F = HIDDEN_DIM // HEADS
FE = F + 1
JB = 256
NJ = NUM_NODES // JB


def _wext_t(W):
    K = W.shape[0]
    Wr = W.reshape(K, HEADS, F).transpose(1, 2, 0)
    return jnp.pad(Wr, ((0, 0), (0, 1), (0, 0))).astype(jnp.float32)


def _dot_t(a, b):
    return jax.lax.dot_general(a, b, (((1,), (1,)), ((), ())),
                               preferred_element_type=jnp.float32)


def _normalize(emb):
    return emb / (jnp.sqrt(jnp.sum(emb * emb, axis=1, keepdims=True)) + 1e-12)


def _fused_body(emb_ref, x_ref, pW_ref, pb_ref,
                W1eT_ref, A1s_ref, A1d_ref, W1as_ref,
                W2eT_ref, A2s_ref, A2d_ref,
                b1_ref, b2_ref, out_ref,
                mask_scr, u1_scr, u2_scr, sc_scr, nd_scr, d1_scr, d2_scr,
                h1_scr, nemb_scr):
    pid = pl.program_id(0)
    r = pid % NJ
    b = r // HEADS
    h = r % HEADS
    ones_row = (jnp.arange(FE) == F).astype(jnp.float32)[:, None]

    def store_factors(whxT, s_row, d_row, s_col):
        smax = jnp.max(s_row)
        dmax = jnp.max(d_row)
        ib, ih = pl.ds(b, 1), pl.ds(h, 1)
        u1_scr[ib, ih] = (jnp.exp(s_row - smax)
                          * whxT).astype(jnp.bfloat16)[None, None]
        u2_scr[ib, ih] = (jnp.exp(0.2 * s_row - smax)
                          * whxT).astype(jnp.bfloat16)[None, None]
        sc_scr[ib, ih] = s_col[None, None]
        nd_scr[ib, ih] = jnp.reshape(-d_row, (NJ, JB))[None, None]
        d1_scr[ib, ih] = jnp.reshape(jnp.exp(d_row - dmax),
                                     (NJ, JB))[None, None]
        d2_scr[ib, ih] = jnp.reshape(jnp.exp(0.2 * d_row - dmax),
                                     (NJ, JB))[None, None]

    def attn(j, first_layer):
        maskT = mask_scr[pl.ds(j, 1)][0]
        zero = jnp.zeros((), jnp.bfloat16)
        jb = pl.ds(j, 1)
        for bb in range(BATCH):
            outs = []
            for hh in range(HEADS):
                cond = sc_scr[bb, hh] >= nd_scr[bb, hh, jb]
                P = jnp.where(cond, maskT, zero)
                Q = maskT - P
                A = jnp.dot(u1_scr[bb, hh], P,
                            preferred_element_type=jnp.float32)
                Bm = jnp.dot(u2_scr[bb, hh], Q,
                             preferred_element_type=jnp.float32)
                R = (d1_scr[bb, hh, jb] * A
                     + d2_scr[bb, hh, jb] * Bm)
                outs.append(R[:F, :] / R[F:FE, :])
            o = jnp.concatenate(outs, axis=0)
            if first_layer:
                o = jnp.maximum(o + b1_ref[...], 0.0)
                h1_scr[pl.ds(j, 1), pl.ds(bb, 1)] = o[None, None]
            else:
                out_ref[bb] = o + b2_ref[...]

    @pl.when(pid < NJ)
    def _phase_mask_prep1():
        @pl.when(pid == 0)
        def _():
            nemb_scr[...] = _normalize(emb_ref[...])
        nf = nemb_scr[...]
        nb = nemb_scr[pl.ds(r * JB, JB), :]
        mask_scr[pl.ds(r, 1)] = (_dot_t(nf, nb) > 0.5).astype(
            jnp.bfloat16)[None]
        hb = jnp.dot(x_ref[pl.ds(b, 1)][0], pW_ref[...],
                     preferred_element_type=jnp.float32) + pb_ref[...][None, :]
        whxT = _dot_t(W1eT_ref[pl.ds(h, 1)][0], hb) + ones_row
        s_row = jnp.dot(A1s_ref[pl.ds(h, 1)][0], whxT,
                        preferred_element_type=jnp.float32)
        d_row = jnp.dot(A1d_ref[pl.ds(h, 1)][0], whxT,
                        preferred_element_type=jnp.float32)
        s_col = jnp.dot(hb, W1as_ref[pl.ds(h, 1)][0],
                        preferred_element_type=jnp.float32)
        store_factors(whxT, s_row, d_row, s_col)

    @pl.when(jnp.logical_and(pid >= NJ, pid < 2 * NJ))
    def _phase_attn1():
        attn(r, True)

    @pl.when(jnp.logical_and(pid >= 2 * NJ, pid < 3 * NJ))
    def _phase_prep2():
        parts = [h1_scr[pl.ds(jj, 1), pl.ds(b, 1)][0, 0] for jj in range(NJ)]
        hbT = jnp.concatenate(parts, axis=1)
        whxT = jnp.dot(W2eT_ref[pl.ds(h, 1)][0], hbT,
                       preferred_element_type=jnp.float32) + ones_row
        s_row = jnp.dot(A2s_ref[pl.ds(h, 1)][0], whxT,
                        preferred_element_type=jnp.float32)
        d_row = jnp.dot(A2d_ref[pl.ds(h, 1)][0], whxT,
                        preferred_element_type=jnp.float32)
        s_col = jnp.reshape(s_row, (NUM_NODES, 1))
        store_factors(whxT, s_row, d_row, s_col)

    @pl.when(pid >= 3 * NJ)
    def _phase_attn2():
        attn(r, False)


def kernel(x, embedding, proj_W, proj_b, W1, a1_src, a1_dst, b1,
           W2, a2_src, a2_dst, b2):
    N = NUM_NODES
    W1eT, W2eT = _wext_t(W1), _wext_t(W2)
    A1s = jnp.pad(a1_src, ((0, 0), (0, 1)))[:, None, :]
    A2s = jnp.pad(a2_src, ((0, 0), (0, 1)))[:, None, :]
    A1d = jnp.pad(a1_dst, ((0, 0), (0, 1)))[:, None, :]
    A2d = jnp.pad(a2_dst, ((0, 0), (0, 1)))[:, None, :]
    W1as = jnp.einsum('khf,hf->hk', W1.reshape(IN_DIM, HEADS, F), a1_src)
    W1as1 = W1as[:, :, None]

    full = lambda *shape: pl.BlockSpec(shape, lambda p: (0,) * len(shape))
    out_t = pl.pallas_call(
        _fused_body,
        grid=(4 * NJ,),
        in_specs=[
            full(N, IN_DIM),
            full(BATCH, N, IN_DIM),
            full(IN_DIM, IN_DIM),
            full(IN_DIM),
            full(HEADS, FE, IN_DIM),
            full(HEADS, 1, FE),
            full(HEADS, 1, FE),
            full(HEADS, IN_DIM, 1),
            full(HEADS, FE, HIDDEN_DIM),
            full(HEADS, 1, FE),
            full(HEADS, 1, FE),
            full(HIDDEN_DIM, 1),
            full(HIDDEN_DIM, 1),
        ],
        out_specs=pl.BlockSpec(
            (BATCH, HIDDEN_DIM, JB),
            lambda p: (0, 0, jnp.maximum(p - 3 * NJ, 0))),
        out_shape=jax.ShapeDtypeStruct((BATCH, HIDDEN_DIM, N), jnp.float32),
        scratch_shapes=[
            pltpu.VMEM((NJ, N, JB), jnp.bfloat16),
            pltpu.VMEM((BATCH, HEADS, FE, N), jnp.bfloat16),
            pltpu.VMEM((BATCH, HEADS, FE, N), jnp.bfloat16),
            pltpu.VMEM((BATCH, HEADS, N, 1), jnp.float32),
            pltpu.VMEM((BATCH, HEADS, NJ, JB), jnp.float32),
            pltpu.VMEM((BATCH, HEADS, NJ, JB), jnp.float32),
            pltpu.VMEM((BATCH, HEADS, NJ, JB), jnp.float32),
            pltpu.VMEM((NJ, BATCH, HIDDEN_DIM, JB), jnp.float32),
            pltpu.VMEM((N, IN_DIM), jnp.float32),
        ],
    )(embedding, x, proj_W, proj_b,
      W1eT, A1s, A1d, W1as1,
      W2eT, A2s, A2d,
      b1[:, None], b2[:, None])
    return jnp.swapaxes(out_t, 1, 2)
